# baseline (speedup 1.0000x reference)
import math

import jax
import jax.numpy as jnp
import numpy as np
from jax import lax
from jax.experimental import pallas as pl
from jax.experimental.pallas import tpu as pltpu

_TOPK = 64
_PRIOR_SCALE = 0.5
_CLIP = 50.0


def _sortable_key_const(x: float) -> int:
    b = np.float32(x).view(np.int32)
    if b < 0:
        b = np.int32(b ^ np.int32(0x7FFFFFFF))
    return int(b)


_LO0 = _sortable_key_const(-_CLIP)
_HI0 = _sortable_key_const(_CLIP) + 1
_BISECT_ITERS = 32


def _fused_body(scale_ref, tg_ref, tfid_t_ref, tfexpr_ref, motif_ref,
                attn_ref, out_ref):
    d = tg_ref.shape[1]
    sim = lax.dot_general(
        tg_ref[...].astype(jnp.bfloat16), tfid_t_ref[...].astype(jnp.bfloat16),
        (((1,), (0,)), ((), ())),
        preferred_element_type=jnp.float32,
    )
    sim = sim / np.float32(math.sqrt(d)) + _PRIOR_SCALE * motif_ref[...]
    sim = jnp.clip(sim, -_CLIP, _CLIP)

    m = jnp.max(sim, axis=1, keepdims=True)
    p = jnp.exp(sim - m)
    z = jnp.sum(p, axis=1, keepdims=True)

    b = lax.bitcast_convert_type(sim, jnp.int32)
    key = jnp.where(b < 0, b ^ jnp.int32(0x7FFFFFFF), b)
    lo = jnp.full((sim.shape[0], 1), _LO0, dtype=jnp.int32)
    hi = jnp.full((sim.shape[0], 1), _HI0, dtype=jnp.int32)

    cnt0 = jnp.full((sim.shape[0], 1), key.shape[1], dtype=jnp.int32)

    def cond(carry):
        i, lo, hi, cl = carry
        return jnp.logical_and(i < _BISECT_ITERS,
                               jnp.logical_not(jnp.all(cl == _TOPK)))

    def bisect(carry):
        i, lo, hi, cl = carry
        mid = (lo & hi) + ((lo ^ hi) >> 1)
        cnt = jnp.sum((key >= mid).astype(jnp.int32), axis=1, keepdims=True)
        ge = cnt >= _TOPK
        return (i + 1, jnp.where(ge, mid, lo), jnp.where(ge, hi, mid),
                jnp.where(ge, cnt, cl))

    _, lo, hi, _ = lax.while_loop(cond, bisect, (0, lo, hi, cnt0))
    mask = key >= lo

    s = jnp.sum(jnp.where(mask, p, 0.0), axis=1, keepdims=True)
    attn = jnp.where(mask, p / (s + 1e-8 * z), 0.0)
    attn_ref[...] = attn

    ts = lax.dot_general(
        attn.astype(jnp.bfloat16), tfexpr_ref[...].astype(jnp.bfloat16),
        (((1,), (1,)), ((), ())),
        preferred_element_type=jnp.float32,
    )
    out_ref[...] = scale_ref[0, 0] * ts


def kernel(tg_emb, tf_id_emb, tf_expr, motif_mask, scale):
    g, d = tg_emb.shape
    t = tf_id_emb.shape[0]
    b = tf_expr.shape[0]
    gb = 1000
    grid = (g // gb,)

    tfid_t = tf_id_emb.T
    scale_arr = jnp.asarray(scale, jnp.float32).reshape(1, 1)

    attn, out_t = pl.pallas_call(
        _fused_body,
        grid=grid,
        in_specs=[
            pl.BlockSpec(memory_space=pltpu.SMEM),
            pl.BlockSpec((gb, d), lambda i: (i, 0)),
            pl.BlockSpec((d, t), lambda i: (0, 0)),
            pl.BlockSpec((b, t), lambda i: (0, 0)),
            pl.BlockSpec((gb, t), lambda i: (i, 0)),
        ],
        out_specs=[
            pl.BlockSpec((gb, t), lambda i: (i, 0)),
            pl.BlockSpec((gb, b), lambda i: (i, 0)),
        ],
        out_shape=[
            jax.ShapeDtypeStruct((g, t), jnp.float32),
            jax.ShapeDtypeStruct((g, b), jnp.float32),
        ],
        compiler_params=pltpu.CompilerParams(
            dimension_semantics=("arbitrary",),
        ),
    )(scale_arr, tg_emb, tfid_t, tf_expr, motif_mask)
    return (out_t.T, attn)
